# initial kernel scaffold (unmeasured)
import jax
import jax.numpy as jnp
from jax import lax
from jax.experimental import pallas as pl
from jax.experimental.pallas import tpu as pltpu


def kernel(
    x,
):
    def body(*refs):
        pass

    out_shape = jax.ShapeDtypeStruct(..., jnp.float32)
    return pl.pallas_call(body, out_shape=out_shape)(...)


# baseline (device time: 298893 ns/iter reference)
import jax
import jax.numpy as jnp
from jax import lax
from jax.experimental import pallas as pl
from jax.experimental.pallas import tpu as pltpu

N_Y = 4
N_STEPS = 2 * (N_Y - 1)


def kernel(x):
    m, n = x.shape
    m_chunk = m // N_Y

    def body(x_ref, out_ref, comm_ref, send_sems, recv_sems):
        my_x = lax.axis_index("x")
        my_y = lax.axis_index("y")
        my_z = lax.axis_index("z")
        right = (my_y + 1) % N_Y
        left = (my_y - 1) % N_Y

        barrier_sem = pltpu.get_barrier_semaphore()
        for nbr in (left, right):
            pl.semaphore_signal(
                barrier_sem,
                inc=1,
                device_id=(my_x, nbr, my_z),
                device_id_type=pl.DeviceIdType.MESH,
            )
        pl.semaphore_wait(barrier_sem, 2)

        out_ref[...] = x_ref[...]

        def chunk(c):
            return pl.ds(c * m_chunk, m_chunk)

        for s in range(N_Y - 1):
            send_c = (my_y - s) % N_Y
            recv_c = (my_y - s - 1) % N_Y
            rdma = pltpu.make_async_remote_copy(
                src_ref=out_ref.at[chunk(send_c)],
                dst_ref=comm_ref.at[s],
                send_sem=send_sems.at[s],
                recv_sem=recv_sems.at[s],
                device_id=(my_x, right, my_z),
                device_id_type=pl.DeviceIdType.MESH,
            )
            rdma.start()
            rdma.wait()
            out_ref[chunk(recv_c), :] += comm_ref[s]

        for s in range(N_Y - 1):
            send_c = (my_y + 1 - s) % N_Y
            rdma = pltpu.make_async_remote_copy(
                src_ref=out_ref.at[chunk(send_c)],
                dst_ref=out_ref.at[chunk(send_c)],
                send_sem=send_sems.at[N_Y - 1 + s],
                recv_sem=recv_sems.at[N_Y - 1 + s],
                device_id=(my_x, right, my_z),
                device_id_type=pl.DeviceIdType.MESH,
            )
            rdma.start()
            rdma.wait()

    return pl.pallas_call(
        body,
        out_shape=jax.ShapeDtypeStruct((m, n), x.dtype),
        in_specs=[pl.BlockSpec(memory_space=pltpu.VMEM)],
        out_specs=pl.BlockSpec(memory_space=pltpu.VMEM),
        scratch_shapes=[
            pltpu.VMEM((N_Y - 1, m_chunk, n), x.dtype),
            pltpu.SemaphoreType.DMA((N_STEPS,)),
            pltpu.SemaphoreType.DMA((N_STEPS,)),
        ],
        compiler_params=pltpu.CompilerParams(collective_id=0),
    )(x)


# device time: 297017 ns/iter; 1.0063x vs baseline; 1.0063x over previous
import jax
import jax.numpy as jnp
from jax import lax
from jax.experimental import pallas as pl
from jax.experimental.pallas import tpu as pltpu

N_Y = 4
N_STEPS = 2 * (N_Y - 1)
N_DIR = 2


def kernel(x):
    m, n = x.shape
    m_half = m // 2
    m_chunk = m_half // N_Y

    def body(x_ref, out_ref, comm_ref, send_sems, recv_sems):
        my_x = lax.axis_index("x")
        my_y = lax.axis_index("y")
        my_z = lax.axis_index("z")
        right = (my_y + 1) % N_Y
        left = (my_y - 1) % N_Y

        barrier_sem = pltpu.get_barrier_semaphore()
        for nbr in (left, right):
            pl.semaphore_signal(
                barrier_sem,
                inc=1,
                device_id=(my_x, nbr, my_z),
                device_id_type=pl.DeviceIdType.MESH,
            )
        pl.semaphore_wait(barrier_sem, 2)

        pos = (my_y, (-my_y) % N_Y)
        dst = ((my_x, right, my_z), (my_x, left, my_z))

        def chunk(d, c):
            return pl.ds(d * m_half + c * m_chunk, m_chunk)

        for s in range(N_Y - 1):
            rdmas = []
            for d in range(N_DIR):
                send_c = (pos[d] - s) % N_Y
                src = x_ref if s == 0 else out_ref
                rdma = pltpu.make_async_remote_copy(
                    src_ref=src.at[chunk(d, send_c)],
                    dst_ref=comm_ref.at[d, s],
                    send_sem=send_sems.at[d, s],
                    recv_sem=recv_sems.at[d, s],
                    device_id=dst[d],
                    device_id_type=pl.DeviceIdType.MESH,
                )
                rdma.start()
                rdmas.append(rdma)
            for d in range(N_DIR):
                rdmas[d].wait()
                recv_c = (pos[d] - s - 1) % N_Y
                out_ref[chunk(d, recv_c), :] = (
                    x_ref[chunk(d, recv_c), :] + comm_ref[d, s]
                )

        for s in range(N_Y - 1):
            rdmas = []
            for d in range(N_DIR):
                send_c = (pos[d] + 1 - s) % N_Y
                rdma = pltpu.make_async_remote_copy(
                    src_ref=out_ref.at[chunk(d, send_c)],
                    dst_ref=out_ref.at[chunk(d, send_c)],
                    send_sem=send_sems.at[d, N_Y - 1 + s],
                    recv_sem=recv_sems.at[d, N_Y - 1 + s],
                    device_id=dst[d],
                    device_id_type=pl.DeviceIdType.MESH,
                )
                rdma.start()
                rdmas.append(rdma)
            for d in range(N_DIR):
                rdmas[d].wait()

    return pl.pallas_call(
        body,
        out_shape=jax.ShapeDtypeStruct((m, n), x.dtype),
        in_specs=[pl.BlockSpec(memory_space=pltpu.VMEM)],
        out_specs=pl.BlockSpec(memory_space=pltpu.VMEM),
        scratch_shapes=[
            pltpu.VMEM((N_DIR, N_Y - 1, m_chunk, n), x.dtype),
            pltpu.SemaphoreType.DMA((N_DIR, N_STEPS)),
            pltpu.SemaphoreType.DMA((N_DIR, N_STEPS)),
        ],
        compiler_params=pltpu.CompilerParams(collective_id=0),
    )(x)


# device time: 211404 ns/iter; 1.4138x vs baseline; 1.4050x over previous
import jax
import jax.numpy as jnp
from jax import lax
from jax.experimental import pallas as pl
from jax.experimental.pallas import tpu as pltpu

N_Y = 4
K = 4


def kernel(x):
    m, n = x.shape
    mc = m // (2 * 2 * K)

    def body(x_ref, out_ref, pbuf, ssy, rsy, ssx, rsx):
        my_x = lax.axis_index("x")
        my_y = lax.axis_index("y")
        my_z = lax.axis_index("z")
        half_off = my_x * (m // 2)

        def dy(y):
            return (my_x, y, my_z)

        xp = (1 - my_x, my_y, my_z)

        def rows(h, k):
            return pl.ds(half_off + (h * K + k) * mc, mc)

        def mkrdma(src, dst, ss, rs, dev):
            return pltpu.make_async_remote_copy(
                src_ref=src,
                dst_ref=dst,
                send_sem=ss,
                recv_sem=rs,
                device_id=dev,
                device_id_type=pl.DeviceIdType.MESH,
            )

        def rwait(rs, sized_ref):
            mkrdma(sized_ref, sized_ref, ssy.at[0, 0], rs, dy(my_y)).wait_recv()

        barrier_sem = pltpu.get_barrier_semaphore()

        @pl.when(my_y > 0)
        def _():
            pl.semaphore_signal(
                barrier_sem, inc=1, device_id=dy(my_y - 1),
                device_id_type=pl.DeviceIdType.MESH,
            )

        @pl.when(my_y < N_Y - 1)
        def _():
            pl.semaphore_signal(
                barrier_sem, inc=1, device_id=dy(my_y + 1),
                device_id_type=pl.DeviceIdType.MESH,
            )

        pl.semaphore_signal(
            barrier_sem, inc=1, device_id=xp,
            device_id_type=pl.DeviceIdType.MESH,
        )
        is_mid = jnp.logical_or(my_y == 1, my_y == 2)

        @pl.when(is_mid)
        def _():
            pl.semaphore_wait(barrier_sem, 3)

        @pl.when(jnp.logical_not(is_mid))
        def _():
            pl.semaphore_wait(barrier_sem, 2)

        def xfwd(h, k):
            r = mkrdma(
                out_ref.at[rows(h, k)], out_ref.at[rows(h, k)],
                ssx.at[h, k], rsx.at[h, k], xp,
            )
            r.start()
            return r

        def end_role(adj, near_h):
            far_h = 1 - near_h
            started = []
            for k in range(K):
                r = mkrdma(x_ref.at[rows(near_h, k)], pbuf.at[0, k],
                           ssy.at[0, k], rsy.at[0, k], dy(adj))
                r.start()
                started.append(r)
                r = mkrdma(x_ref.at[rows(far_h, k)], pbuf.at[2, k],
                           ssy.at[1, k], rsy.at[2, k], dy(adj))
                r.start()
                started.append(r)
            for k in range(K):
                rwait(rsy.at[0, k], out_ref.at[rows(near_h, k)])
                started.append(xfwd(near_h, k))
            for k in range(K):
                rwait(rsy.at[1, k], out_ref.at[rows(far_h, k)])
                started.append(xfwd(far_h, k))
            return started

        def comb_role(end_y, other_y, near_h):
            far_h = 1 - near_h
            started = []
            for k in range(K):
                rwait(rsy.at[2, k], pbuf.at[2, k])
                out_ref[rows(far_h, k), :] = x_ref[rows(far_h, k), :] + pbuf[2, k]
                r = mkrdma(out_ref.at[rows(far_h, k)], pbuf.at[1, k],
                           ssy.at[0, k], rsy.at[1, k], dy(other_y))
                r.start()
                started.append(r)
            for k in range(K):
                rwait(rsy.at[0, k], pbuf.at[0, k])
                rwait(rsy.at[1, k], pbuf.at[1, k])
                out_ref[rows(near_h, k), :] = (
                    x_ref[rows(near_h, k), :] + pbuf[0, k] + pbuf[1, k]
                )
                r = mkrdma(out_ref.at[rows(near_h, k)],
                           out_ref.at[rows(near_h, k)],
                           ssy.at[1, k], rsy.at[0, k], dy(end_y))
                r.start()
                started.append(r)
                r = mkrdma(out_ref.at[rows(near_h, k)],
                           out_ref.at[rows(near_h, k)],
                           ssy.at[2, k], rsy.at[3, k], dy(other_y))
                r.start()
                started.append(r)
                started.append(xfwd(near_h, k))
            for k in range(K):
                rwait(rsy.at[3, k], out_ref.at[rows(far_h, k)])
                r = mkrdma(out_ref.at[rows(far_h, k)],
                           out_ref.at[rows(far_h, k)],
                           ssy.at[3, k], rsy.at[1, k], dy(end_y))
                r.start()
                started.append(r)
                started.append(xfwd(far_h, k))
            return started

        def finish(started):
            for k in range(K):
                rwait(rsx.at[0, k], pbuf.at[0, k])
                rwait(rsx.at[1, k], pbuf.at[1, k])
            for r in started:
                r.wait_send()

        @pl.when(my_y == 0)
        def _():
            finish(end_role(1, 0))

        @pl.when(my_y == 1)
        def _():
            finish(comb_role(0, 2, 0))

        @pl.when(my_y == 2)
        def _():
            finish(comb_role(3, 1, 1))

        @pl.when(my_y == 3)
        def _():
            finish(end_role(2, 1))

    return pl.pallas_call(
        body,
        out_shape=jax.ShapeDtypeStruct((m, n), x.dtype),
        in_specs=[pl.BlockSpec(memory_space=pltpu.VMEM)],
        out_specs=pl.BlockSpec(memory_space=pltpu.VMEM),
        scratch_shapes=[
            pltpu.VMEM((3, K, mc, n), x.dtype),
            pltpu.SemaphoreType.DMA((4, K)),
            pltpu.SemaphoreType.DMA((4, K)),
            pltpu.SemaphoreType.DMA((2, K)),
            pltpu.SemaphoreType.DMA((2, K)),
        ],
        compiler_params=pltpu.CompilerParams(collective_id=0),
    )(x)


# device time: 200457 ns/iter; 1.4911x vs baseline; 1.0546x over previous
import jax
import jax.numpy as jnp
from jax import lax
from jax.experimental import pallas as pl
from jax.experimental.pallas import tpu as pltpu

N_Y = 4
K = 8


def kernel(x):
    m, n = x.shape
    mc = m // (2 * 2 * K)

    def body(x_ref, out_ref, pbuf, ssy, rsy, ssx, rsx):
        my_x = lax.axis_index("x")
        my_y = lax.axis_index("y")
        my_z = lax.axis_index("z")
        half_off = my_x * (m // 2)

        def dy(y):
            return (my_x, y, my_z)

        xp = (1 - my_x, my_y, my_z)

        def rows(h, k):
            return pl.ds(half_off + (h * K + k) * mc, mc)

        def mkrdma(src, dst, ss, rs, dev):
            return pltpu.make_async_remote_copy(
                src_ref=src,
                dst_ref=dst,
                send_sem=ss,
                recv_sem=rs,
                device_id=dev,
                device_id_type=pl.DeviceIdType.MESH,
            )

        def rwait(rs, sized_ref):
            mkrdma(sized_ref, sized_ref, ssy.at[0, 0], rs, dy(my_y)).wait_recv()

        barrier_sem = pltpu.get_barrier_semaphore()

        @pl.when(my_y > 0)
        def _():
            pl.semaphore_signal(
                barrier_sem, inc=1, device_id=dy(my_y - 1),
                device_id_type=pl.DeviceIdType.MESH,
            )

        @pl.when(my_y < N_Y - 1)
        def _():
            pl.semaphore_signal(
                barrier_sem, inc=1, device_id=dy(my_y + 1),
                device_id_type=pl.DeviceIdType.MESH,
            )

        pl.semaphore_signal(
            barrier_sem, inc=1, device_id=xp,
            device_id_type=pl.DeviceIdType.MESH,
        )
        is_mid = jnp.logical_or(my_y == 1, my_y == 2)

        @pl.when(is_mid)
        def _():
            pl.semaphore_wait(barrier_sem, 3)

        @pl.when(jnp.logical_not(is_mid))
        def _():
            pl.semaphore_wait(barrier_sem, 2)

        def xfwd(h, k):
            r = mkrdma(
                out_ref.at[rows(h, k)], out_ref.at[rows(h, k)],
                ssx.at[h, k], rsx.at[h, k], xp,
            )
            r.start()
            return r

        def end_role(adj, near_h):
            far_h = 1 - near_h
            started = []
            for k in range(K):
                r = mkrdma(x_ref.at[rows(far_h, k)], pbuf.at[2, k],
                           ssy.at[1, k], rsy.at[2, k], dy(adj))
                r.start()
                started.append(r)
                r = mkrdma(x_ref.at[rows(near_h, k)], pbuf.at[0, k],
                           ssy.at[0, k], rsy.at[0, k], dy(adj))
                r.start()
                started.append(r)
            for k in range(K):
                rwait(rsy.at[0, k], out_ref.at[rows(near_h, k)])
                started.append(xfwd(near_h, k))
            for k in range(K):
                rwait(rsy.at[1, k], out_ref.at[rows(far_h, k)])
                started.append(xfwd(far_h, k))
            return started

        def comb_role(end_y, other_y, near_h):
            far_h = 1 - near_h
            started = []
            for k in range(K):
                rwait(rsy.at[2, k], pbuf.at[2, k])
                out_ref[rows(far_h, k), :] = x_ref[rows(far_h, k), :] + pbuf[2, k]
                r = mkrdma(out_ref.at[rows(far_h, k)], pbuf.at[1, k],
                           ssy.at[0, k], rsy.at[1, k], dy(other_y))
                r.start()
                started.append(r)
            for k in range(K):
                rwait(rsy.at[0, k], pbuf.at[0, k])
                rwait(rsy.at[1, k], pbuf.at[1, k])
                out_ref[rows(near_h, k), :] = (
                    x_ref[rows(near_h, k), :] + pbuf[0, k] + pbuf[1, k]
                )
                r = mkrdma(out_ref.at[rows(near_h, k)],
                           out_ref.at[rows(near_h, k)],
                           ssy.at[1, k], rsy.at[0, k], dy(end_y))
                r.start()
                started.append(r)
                r = mkrdma(out_ref.at[rows(near_h, k)],
                           out_ref.at[rows(near_h, k)],
                           ssy.at[2, k], rsy.at[3, k], dy(other_y))
                r.start()
                started.append(r)
                started.append(xfwd(near_h, k))
            for k in range(K):
                rwait(rsy.at[3, k], out_ref.at[rows(far_h, k)])
                r = mkrdma(out_ref.at[rows(far_h, k)],
                           out_ref.at[rows(far_h, k)],
                           ssy.at[3, k], rsy.at[1, k], dy(end_y))
                r.start()
                started.append(r)
                started.append(xfwd(far_h, k))
            return started

        def finish(started):
            for k in range(K):
                rwait(rsx.at[0, k], pbuf.at[0, k])
                rwait(rsx.at[1, k], pbuf.at[1, k])
            for r in started:
                r.wait_send()

        @pl.when(my_y == 0)
        def _():
            finish(end_role(1, 0))

        @pl.when(my_y == 1)
        def _():
            finish(comb_role(0, 2, 0))

        @pl.when(my_y == 2)
        def _():
            finish(comb_role(3, 1, 1))

        @pl.when(my_y == 3)
        def _():
            finish(end_role(2, 1))

    return pl.pallas_call(
        body,
        out_shape=jax.ShapeDtypeStruct((m, n), x.dtype),
        in_specs=[pl.BlockSpec(memory_space=pltpu.VMEM)],
        out_specs=pl.BlockSpec(memory_space=pltpu.VMEM),
        scratch_shapes=[
            pltpu.VMEM((3, K, mc, n), x.dtype),
            pltpu.SemaphoreType.DMA((4, K)),
            pltpu.SemaphoreType.DMA((4, K)),
            pltpu.SemaphoreType.DMA((2, K)),
            pltpu.SemaphoreType.DMA((2, K)),
        ],
        compiler_params=pltpu.CompilerParams(collective_id=0),
    )(x)


# device time: 187430 ns/iter; 1.5947x vs baseline; 1.0695x over previous
import jax
import jax.numpy as jnp
from jax import lax
from jax.experimental import pallas as pl
from jax.experimental.pallas import tpu as pltpu

N_Y = 4
N_Z = 4
K = 1


def kernel(x):
    m, n = x.shape
    pr = m // (2 * N_Z)
    mc = pr // (2 * K)

    def body(x_ref, out_ref, pbuf, ssy, rsy, ssz, rsz, ssx, rsx):
        my_x = lax.axis_index("x")
        my_y = lax.axis_index("y")
        my_z = lax.axis_index("z")
        part_off = (my_x * N_Z + my_z) * pr

        def dy(y):
            return (my_x, y, my_z)

        def dz(z):
            return (my_x, my_y, z)

        xp = (1 - my_x, my_y, my_z)

        def rows(h, k):
            return pl.ds(part_off + (h * K + k) * mc, mc)

        def prow(xi, zo):
            return pl.ds((xi * N_Z + zo) * pr, pr)

        def mkrdma(src, dst, ss, rs, dev):
            return pltpu.make_async_remote_copy(
                src_ref=src,
                dst_ref=dst,
                send_sem=ss,
                recv_sem=rs,
                device_id=dev,
                device_id_type=pl.DeviceIdType.MESH,
            )

        def rwait(rs, sized_ref):
            mkrdma(sized_ref, sized_ref, ssy.at[0, 0], rs, dy(my_y)).wait_recv()

        barrier_sem = pltpu.get_barrier_semaphore()

        @pl.when(my_y > 0)
        def _():
            pl.semaphore_signal(
                barrier_sem, inc=1, device_id=dy(my_y - 1),
                device_id_type=pl.DeviceIdType.MESH,
            )

        @pl.when(my_y < N_Y - 1)
        def _():
            pl.semaphore_signal(
                barrier_sem, inc=1, device_id=dy(my_y + 1),
                device_id_type=pl.DeviceIdType.MESH,
            )

        @pl.when(my_z > 0)
        def _():
            pl.semaphore_signal(
                barrier_sem, inc=1, device_id=dz(my_z - 1),
                device_id_type=pl.DeviceIdType.MESH,
            )

        @pl.when(my_z < N_Z - 1)
        def _():
            pl.semaphore_signal(
                barrier_sem, inc=1, device_id=dz(my_z + 1),
                device_id_type=pl.DeviceIdType.MESH,
            )

        pl.semaphore_signal(
            barrier_sem, inc=1, device_id=xp,
            device_id_type=pl.DeviceIdType.MESH,
        )
        mid_y = jnp.logical_or(my_y == 1, my_y == 2)
        mid_z = jnp.logical_or(my_z == 1, my_z == 2)
        for my, mz, cnt in ((0, 0, 3), (1, 0, 4), (0, 1, 4), (1, 1, 5)):
            @pl.when(jnp.logical_and(mid_y == my, mid_z == mz))
            def _(cnt=cnt):
                pl.semaphore_wait(barrier_sem, cnt)

        def end_role(adj, near_h):
            far_h = 1 - near_h
            started = []
            for k in range(K):
                r = mkrdma(x_ref.at[rows(far_h, k)], pbuf.at[2, k],
                           ssy.at[1, k], rsy.at[2, k], dy(adj))
                r.start()
                started.append(r)
                r = mkrdma(x_ref.at[rows(near_h, k)], pbuf.at[0, k],
                           ssy.at[0, k], rsy.at[0, k], dy(adj))
                r.start()
                started.append(r)
            for k in range(K):
                rwait(rsy.at[0, k], out_ref.at[rows(near_h, k)])
            for k in range(K):
                rwait(rsy.at[1, k], out_ref.at[rows(far_h, k)])
            return started

        def comb_role(end_y, other_y, near_h):
            far_h = 1 - near_h
            started = []
            for k in range(K):
                rwait(rsy.at[2, k], pbuf.at[2, k])
                out_ref[rows(far_h, k), :] = x_ref[rows(far_h, k), :] + pbuf[2, k]
                r = mkrdma(out_ref.at[rows(far_h, k)], pbuf.at[1, k],
                           ssy.at[0, k], rsy.at[1, k], dy(other_y))
                r.start()
                started.append(r)
            for k in range(K):
                rwait(rsy.at[0, k], pbuf.at[0, k])
                rwait(rsy.at[1, k], pbuf.at[1, k])
                out_ref[rows(near_h, k), :] = (
                    x_ref[rows(near_h, k), :] + pbuf[0, k] + pbuf[1, k]
                )
                r = mkrdma(out_ref.at[rows(near_h, k)],
                           out_ref.at[rows(near_h, k)],
                           ssy.at[1, k], rsy.at[0, k], dy(end_y))
                r.start()
                started.append(r)
                r = mkrdma(out_ref.at[rows(near_h, k)],
                           out_ref.at[rows(near_h, k)],
                           ssy.at[2, k], rsy.at[3, k], dy(other_y))
                r.start()
                started.append(r)
            for k in range(K):
                rwait(rsy.at[3, k], out_ref.at[rows(far_h, k)])
                r = mkrdma(out_ref.at[rows(far_h, k)],
                           out_ref.at[rows(far_h, k)],
                           ssy.at[3, k], rsy.at[1, k], dy(end_y))
                r.start()
                started.append(r)
            return started

        def drain(started):
            for r in started:
                r.wait_send()

        @pl.when(my_y == 0)
        def _():
            drain(end_role(1, 0))

        @pl.when(my_y == 1)
        def _():
            drain(comb_role(0, 2, 0))

        @pl.when(my_y == 2)
        def _():
            drain(comb_role(3, 1, 1))

        @pl.when(my_y == 3)
        def _():
            drain(end_role(2, 1))

        def zsend(zo, direction, to_z):
            r = mkrdma(out_ref.at[prow(my_x, zo)], out_ref.at[prow(my_x, zo)],
                       ssz.at[direction, zo], rsz.at[direction, zo], dz(to_z))
            r.start()
            return r

        def xsend(zo):
            r = mkrdma(out_ref.at[prow(my_x, zo)], out_ref.at[prow(my_x, zo)],
                       ssx.at[zo], rsx.at[zo], xp)
            r.start()
            return r

        def z_role(zc):
            started = []
            if zc < N_Z - 1:
                started.append(zsend(zc, 0, zc + 1))
            if zc > 0:
                started.append(zsend(zc, 1, zc - 1))
            started.append(xsend(zc))
            for zo in range(zc - 1, -1, -1):
                rwait(rsz.at[0, zo], out_ref.at[prow(my_x, zo)])
                if zc < N_Z - 1:
                    started.append(zsend(zo, 0, zc + 1))
                started.append(xsend(zo))
            for zo in range(zc + 1, N_Z):
                rwait(rsz.at[1, zo], out_ref.at[prow(my_x, zo)])
                if zc > 0:
                    started.append(zsend(zo, 1, zc - 1))
                started.append(xsend(zo))
            return started

        for zc in range(N_Z):
            @pl.when(my_z == zc)
            def _(zc=zc):
                drain(z_role(zc))

        rwait(rsx.at[my_z], out_ref.at[prow(1 - my_x, my_z)])
        for d in range(1, N_Z):
            @pl.when(my_z - d >= 0)
            def _(d=d):
                rwait(rsx.at[my_z - d], out_ref.at[prow(1 - my_x, my_z - d)])

            @pl.when(my_z + d <= N_Z - 1)
            def _(d=d):
                rwait(rsx.at[my_z + d], out_ref.at[prow(1 - my_x, my_z + d)])

    return pl.pallas_call(
        body,
        out_shape=jax.ShapeDtypeStruct((m, n), x.dtype),
        in_specs=[pl.BlockSpec(memory_space=pltpu.VMEM)],
        out_specs=pl.BlockSpec(memory_space=pltpu.VMEM),
        scratch_shapes=[
            pltpu.VMEM((3, K, mc, n), x.dtype),
            pltpu.SemaphoreType.DMA((4, K)),
            pltpu.SemaphoreType.DMA((4, K)),
            pltpu.SemaphoreType.DMA((2, N_Z)),
            pltpu.SemaphoreType.DMA((2, N_Z)),
            pltpu.SemaphoreType.DMA((N_Z,)),
            pltpu.SemaphoreType.DMA((N_Z,)),
        ],
        compiler_params=pltpu.CompilerParams(collective_id=0),
    )(x)
